# Initial kernel scaffold; baseline (speedup 1.0000x reference)
import functools

import jax
import jax.numpy as jnp
from jax import lax
from jax.experimental import pallas as pl
from jax.experimental.pallas import tpu as pltpu
from jax.experimental.pallas import tpu_sc as plsc

F32 = jnp.float32
I32 = jnp.int32
D = 64
NC, NS, NW = 2, 16, 32
EB = 128
BR = 512

N = 10000
E = 160000
NE = 100000
B = 256

NR = 10240
DN = 10000
ANR = 10752
ADN = 10256
EP = 163840
EAP = 184320
NEP = 102400
ESP = 81920
SR = 272


def _mesh():
    return plsc.VectorSubcoreMesh(core_axis_name="c", subcore_axis_name="s")


def _wid():
    return lax.axis_index("s") * NC + lax.axis_index("c")


@functools.lru_cache(None)
def _build_gather3(ep, nr):
    pt = ep // NW
    nb = pt // EB

    def body(q_h, k_h, v_h, src_h, dst_h, qd_h, ks_h, vs_h,
             is_v, id_v, bq, bk, bv, s1, s2, s3):
        base = _wid() * pt

        def blk(b, carry):
            off = base + b * EB
            pltpu.sync_copy(src_h.at[pl.ds(off, EB)], is_v)
            pltpu.sync_copy(dst_h.at[pl.ds(off, EB)], id_v)
            d1 = pltpu.async_copy(q_h.at[id_v], bq, s1)
            d2 = pltpu.async_copy(k_h.at[is_v], bk, s2)
            d3 = pltpu.async_copy(v_h.at[is_v], bv, s3)
            d1.wait()
            d2.wait()
            d3.wait()
            w1 = pltpu.async_copy(bq, qd_h.at[pl.ds(off, EB)], s1)
            w2 = pltpu.async_copy(bk, ks_h.at[pl.ds(off, EB)], s2)
            w3 = pltpu.async_copy(bv, vs_h.at[pl.ds(off, EB)], s3)
            w1.wait()
            w2.wait()
            w3.wait()
            return carry

        lax.fori_loop(0, nb, blk, 0)

    return pl.kernel(
        body,
        out_type=(jax.ShapeDtypeStruct((ep, D), F32),) * 3,
        mesh=_mesh(),
        scratch_types=[
            pltpu.VMEM((EB,), I32), pltpu.VMEM((EB,), I32),
            pltpu.VMEM((EB, D), F32), pltpu.VMEM((EB, D), F32),
            pltpu.VMEM((EB, D), F32),
            pltpu.SemaphoreType.DMA, pltpu.SemaphoreType.DMA,
            pltpu.SemaphoreType.DMA,
        ],
        name=f"sc_gather3_{ep}",
    )


@functools.lru_cache(None)
def _build_segmax(ep, nr):
    pt = ep // NW
    nb = pt // EB

    def body(lg_h, dst_h, ninf_h, mp_h, m_loc, id_v, lg_v):
        w = _wid()
        pltpu.sync_copy(ninf_h, m_loc)
        base = w * pt

        def blk(b, carry):
            off = base + b * EB
            pltpu.sync_copy(dst_h.at[pl.ds(off, EB)], id_v)
            pltpu.sync_copy(lg_h.at[pl.ds(off, EB)], lg_v)
            for g in range(EB // 16):
                d16 = id_v[pl.ds(g * 16, 16)]
                l16 = lg_v[pl.ds(g * 16, 16)]
                cur0 = plsc.load_gather(m_loc, [d16])

                def cond(cur):
                    return jnp.any(cur < l16)

                def bd(cur):
                    plsc.store_scatter(m_loc, [d16], jnp.maximum(cur, l16))
                    return plsc.load_gather(m_loc, [d16])

                lax.while_loop(cond, bd, cur0)
            return carry

        lax.fori_loop(0, nb, blk, 0)
        pltpu.sync_copy(m_loc, mp_h.at[w])

    return pl.kernel(
        body,
        out_type=jax.ShapeDtypeStruct((NW, nr), F32),
        mesh=_mesh(),
        scratch_types=[
            pltpu.VMEM((nr,), F32), pltpu.VMEM((EB,), I32),
            pltpu.VMEM((EB,), F32),
        ],
        name=f"sc_segmax_{ep}",
    )


@functools.lru_cache(None)
def _build_tscatter(ep, nr, has_ee):
    pt = ep // NW
    nb = pt // EB
    zsl = nr // NS

    def body(*refs):
        if has_ee:
            (lg_h, dst_h, m_h, vs_h, ee_h, z2_h, z1_h, agg_h, sp_h,
             agg_sh, m_loc, s_loc, id_v, lg_v, bv, be, bo, s1, s2) = refs
        else:
            (lg_h, dst_h, m_h, vs_h, z2_h, z1_h, agg_h, sp_h,
             agg_sh, m_loc, s_loc, id_v, lg_v, bv, bo, s1, s2) = refs
            be = None
        c = lax.axis_index("c")
        s = lax.axis_index("s")
        w = s * NC + c
        pltpu.sync_copy(m_h, m_loc)
        pltpu.sync_copy(z1_h, s_loc)
        pltpu.sync_copy(z2_h.at[pl.ds(s * zsl, zsl)],
                        agg_sh.at[pl.ds(s * zsl, zsl)])
        plsc.subcore_barrier()
        base = w * pt

        def blk(b, carry):
            off = base + b * EB
            pltpu.sync_copy(dst_h.at[pl.ds(off, EB)], id_v)
            pltpu.sync_copy(lg_h.at[pl.ds(off, EB)], lg_v)
            d1 = pltpu.async_copy(vs_h.at[pl.ds(off, EB)], bv, s1)
            if has_ee:
                d2 = pltpu.async_copy(ee_h.at[pl.ds(off, EB)], be, s2)
            d1.wait()
            if has_ee:
                d2.wait()
            for g in range(EB // 16):
                d16 = id_v[pl.ds(g * 16, 16)]
                l16 = lg_v[pl.ds(g * 16, 16)]
                mv = plsc.load_gather(m_loc, [d16])
                ex = jnp.exp(l16 - mv)
                plsc.addupdate_scatter(s_loc, [d16], ex)
                for j in range(16):
                    ei = g * 16 + j
                    exj = ex[j]
                    for ch in range(4):
                        sl = pl.ds(ch * 16, 16)
                        vv = bv[ei, sl]
                        if has_ee:
                            vv = vv + be[ei, sl]
                        bo[ei, sl] = vv * exj
            pltpu.sync_copy(bo, agg_sh.at[id_v], add=True)
            return carry

        lax.fori_loop(0, nb, blk, 0)
        plsc.subcore_barrier()
        pltpu.sync_copy(agg_sh.at[pl.ds(s * zsl, zsl)],
                        agg_h.at[c, pl.ds(s * zsl, zsl)])
        pltpu.sync_copy(s_loc, sp_h.at[w])

    scratch = [
        pltpu.VMEM_SHARED((nr, D), F32),
        pltpu.VMEM((nr,), F32), pltpu.VMEM((nr,), F32),
        pltpu.VMEM((EB,), I32), pltpu.VMEM((EB,), F32),
        pltpu.VMEM((EB, D), F32),
    ]
    if has_ee:
        scratch.append(pltpu.VMEM((EB, D), F32))
    scratch.append(pltpu.VMEM((EB, D), F32))
    scratch += [pltpu.SemaphoreType.DMA, pltpu.SemaphoreType.DMA]
    return pl.kernel(
        body,
        out_type=(jax.ShapeDtypeStruct((NC, nr, D), F32),
                  jax.ShapeDtypeStruct((NW, nr), F32)),
        mesh=_mesh(),
        scratch_types=scratch,
        name=f"sc_tscatter_{ep}_{int(has_ee)}",
    )


@functools.lru_cache(None)
def _build_gscatter(ep, nr):
    pt = ep // NW
    nb = pt // EB
    zsl = nr // NS

    def body(o_h, e_h, src_h, dst_h, z2_h, agg_h,
             agg_sh, is_v, id_v, bo, be, bm, s1, s2):
        c = lax.axis_index("c")
        s = lax.axis_index("s")
        w = s * NC + c
        pltpu.sync_copy(z2_h.at[pl.ds(s * zsl, zsl)],
                        agg_sh.at[pl.ds(s * zsl, zsl)])
        plsc.subcore_barrier()
        base = w * pt

        def blk(b, carry):
            off = base + b * EB
            pltpu.sync_copy(src_h.at[pl.ds(off, EB)], is_v)
            pltpu.sync_copy(dst_h.at[pl.ds(off, EB)], id_v)
            d1 = pltpu.async_copy(o_h.at[is_v], bo, s1)
            d2 = pltpu.async_copy(e_h.at[pl.ds(off, EB)], be, s2)
            d1.wait()
            d2.wait()
            for r in range(EB):
                for ch in range(4):
                    sl = pl.ds(ch * 16, 16)
                    bm[r, sl] = jnp.maximum(bo[r, sl] + be[r, sl], 0.0) + 1e-7
            pltpu.sync_copy(bm, agg_sh.at[id_v], add=True)
            return carry

        lax.fori_loop(0, nb, blk, 0)
        plsc.subcore_barrier()
        pltpu.sync_copy(agg_sh.at[pl.ds(s * zsl, zsl)],
                        agg_h.at[c, pl.ds(s * zsl, zsl)])

    return pl.kernel(
        body,
        out_type=jax.ShapeDtypeStruct((NC, nr, D), F32),
        mesh=_mesh(),
        scratch_types=[
            pltpu.VMEM_SHARED((nr, D), F32),
            pltpu.VMEM((EB,), I32), pltpu.VMEM((EB,), I32),
            pltpu.VMEM((EB, D), F32), pltpu.VMEM((EB, D), F32),
            pltpu.VMEM((EB, D), F32),
            pltpu.SemaphoreType.DMA, pltpu.SemaphoreType.DMA,
        ],
        name=f"sc_gscatter_{ep}",
    )


@functools.lru_cache(None)
def _build_pairadd(ep, nr):
    pt = ep // NW
    nb = pt // EB

    def body(t_h, ia_h, ib_h, out_h, ia_v, ib_v, ba, bb, s1, s2):
        base = _wid() * pt

        def blk(b, carry):
            off = base + b * EB
            pltpu.sync_copy(ia_h.at[pl.ds(off, EB)], ia_v)
            pltpu.sync_copy(ib_h.at[pl.ds(off, EB)], ib_v)
            d1 = pltpu.async_copy(t_h.at[ia_v], ba, s1)
            d2 = pltpu.async_copy(t_h.at[ib_v], bb, s2)
            d1.wait()
            d2.wait()
            for r in range(EB):
                for ch in range(4):
                    sl = pl.ds(ch * 16, 16)
                    ba[r, sl] = ba[r, sl] + bb[r, sl]
            pltpu.sync_copy(ba, out_h.at[pl.ds(off, EB)])
            return carry

        lax.fori_loop(0, nb, blk, 0)

    return pl.kernel(
        body,
        out_type=jax.ShapeDtypeStruct((ep, D), F32),
        mesh=_mesh(),
        scratch_types=[
            pltpu.VMEM((EB,), I32), pltpu.VMEM((EB,), I32),
            pltpu.VMEM((EB, D), F32), pltpu.VMEM((EB, D), F32),
            pltpu.SemaphoreType.DMA, pltpu.SemaphoreType.DMA,
        ],
        name=f"sc_pairadd_{ep}",
    )


@functools.lru_cache(None)
def _build_pool(pr):
    pt = pr // NW
    pb = 112
    nb = pt // pb
    zsl = SR // NS

    def body(rows_h, pidx_h, z2_h, z1_h, sums_h, cnt_h,
             sums_sh, cnt_loc, ip_v, br_v, s1):
        c = lax.axis_index("c")
        s = lax.axis_index("s")
        w = s * NC + c
        pltpu.sync_copy(z1_h, cnt_loc)
        pltpu.sync_copy(z2_h.at[pl.ds(s * zsl, zsl)],
                        sums_sh.at[pl.ds(s * zsl, zsl)])
        plsc.subcore_barrier()
        base = w * pt
        ones = jnp.ones((16,), F32)

        def blk(b, carry):
            off = base + b * pb
            pltpu.sync_copy(pidx_h.at[pl.ds(off, pb)], ip_v)
            d1 = pltpu.async_copy(rows_h.at[pl.ds(off, pb)], br_v, s1)
            d1.wait()
            for g in range(pb // 16):
                p16 = ip_v[pl.ds(g * 16, 16)]
                plsc.addupdate_scatter(cnt_loc, [p16], ones)
            pltpu.sync_copy(br_v, sums_sh.at[ip_v], add=True)
            return carry

        lax.fori_loop(0, nb, blk, 0)
        plsc.subcore_barrier()
        pltpu.sync_copy(sums_sh.at[pl.ds(s * zsl, zsl)],
                        sums_h.at[c, pl.ds(s * zsl, zsl)])
        pltpu.sync_copy(cnt_loc, cnt_h.at[w])

    return pl.kernel(
        body,
        out_type=(jax.ShapeDtypeStruct((NC, SR, D), F32),
                  jax.ShapeDtypeStruct((NW, SR), F32)),
        mesh=_mesh(),
        scratch_types=[
            pltpu.VMEM_SHARED((SR, D), F32),
            pltpu.VMEM((SR,), F32), pltpu.VMEM((112,), I32),
            pltpu.VMEM((112, D), F32),
            pltpu.SemaphoreType.DMA,
        ],
        name="sc_pool",
    )


def _lin_tc(x, w, b):
    r, din = x.shape
    dout = w.shape[1]
    br = min(BR, r)

    def body(x_r, w_r, b_r, o_r):
        o_r[...] = jnp.dot(x_r[...], w_r[...],
                           preferred_element_type=F32) + b_r[...]

    return pl.pallas_call(
        body, grid=(r // br,),
        in_specs=[pl.BlockSpec((br, din), lambda i: (i, 0)),
                  pl.BlockSpec((din, dout), lambda i: (0, 0)),
                  pl.BlockSpec((1, dout), lambda i: (0, 0))],
        out_specs=pl.BlockSpec((br, dout), lambda i: (i, 0)),
        out_shape=jax.ShapeDtypeStruct((r, dout), F32),
    )(x, w, b.reshape(1, -1))


def _qkvs_tc(o, p):
    r = o.shape[0]

    def body(o_r, wq, bq, wk, bk, wv, bv, ws, bs, q_o, k_o, v_o, s_o):
        ob = o_r[...]
        q_o[...] = jnp.dot(ob, wq[...], preferred_element_type=F32) + bq[...]
        k_o[...] = jnp.dot(ob, wk[...], preferred_element_type=F32) + bk[...]
        v_o[...] = jnp.dot(ob, wv[...], preferred_element_type=F32) + bv[...]
        s_o[...] = jnp.dot(ob, ws[...], preferred_element_type=F32) + bs[...]

    wspec = pl.BlockSpec((D, D), lambda i: (0, 0))
    bspec = pl.BlockSpec((1, D), lambda i: (0, 0))
    rspec = pl.BlockSpec((BR, D), lambda i: (i, 0))
    return pl.pallas_call(
        body, grid=(r // BR,),
        in_specs=[rspec, wspec, bspec, wspec, bspec, wspec, bspec, wspec,
                  bspec],
        out_specs=[rspec] * 4,
        out_shape=[jax.ShapeDtypeStruct((r, D), F32)] * 4,
    )(o, p["q"]["W"], p["q"]["b"].reshape(1, D),
      p["k"]["W"], p["k"]["b"].reshape(1, D),
      p["v"]["W"], p["v"]["b"].reshape(1, D),
      p["skip"]["W"], p["skip"]["b"].reshape(1, D))


def _logits_tc(qd, ks, ee):
    r = qd.shape[0]
    if ee is None:
        def body(q_r, k_r, o_r):
            o_r[...] = jnp.sum(q_r[...] * k_r[...], axis=1,
                               keepdims=True) * 0.125
        args = (qd, ks)
        nin = 2
    else:
        def body(q_r, k_r, e_r, o_r):
            o_r[...] = jnp.sum(q_r[...] * (k_r[...] + e_r[...]), axis=1,
                               keepdims=True) * 0.125
        args = (qd, ks, ee)
        nin = 3
    out = pl.pallas_call(
        body, grid=(r // BR,),
        in_specs=[pl.BlockSpec((BR, D), lambda i: (i, 0))] * nin,
        out_specs=pl.BlockSpec((BR, 1), lambda i: (i, 0)),
        out_shape=jax.ShapeDtypeStruct((r, 1), F32),
    )(*args)
    return out.reshape(r)


def _maxred_tc(mp):
    nw, nr = mp.shape

    def body(m_r, o_r):
        o_r[...] = jnp.max(m_r[...], axis=0, keepdims=True)

    out = pl.pallas_call(
        body, grid=(nr // BR,),
        in_specs=[pl.BlockSpec((nw, BR), lambda i: (0, i))],
        out_specs=pl.BlockSpec((1, BR), lambda i: (0, i)),
        out_shape=jax.ShapeDtypeStruct((1, nr), F32),
    )(mp)
    return out.reshape(nr)


def _combine_tc(aggp, sp, sk, o_prev=None):
    nr = sk.shape[0]
    resid = o_prev is not None

    def body(*refs):
        if resid:
            a_r, s_r, k_r, p_r, o_r = refs
        else:
            a_r, s_r, k_r, o_r = refs
        agg = a_r[0] + a_r[1]
        s = jnp.sum(s_r[...], axis=0)
        res = agg / (s[:, None] + 1e-16) + k_r[...]
        if resid:
            res = res + p_r[...]
        o_r[...] = res

    specs = [pl.BlockSpec((NC, BR, D), lambda i: (0, i, 0)),
             pl.BlockSpec((NW, BR), lambda i: (0, i)),
             pl.BlockSpec((BR, D), lambda i: (i, 0))]
    args = [aggp, sp, sk]
    if resid:
        specs.append(pl.BlockSpec((BR, D), lambda i: (i, 0)))
        args.append(o_prev)
    return pl.pallas_call(
        body, grid=(nr // BR,),
        in_specs=specs,
        out_specs=pl.BlockSpec((BR, D), lambda i: (i, 0)),
        out_shape=jax.ShapeDtypeStruct((nr, D), F32),
    )(*args)


def _gencomb_tc(aggp, o, p):
    nr = o.shape[0]

    def body(a_r, o_r, w_r, b_r, out_r):
        xx = a_r[0] + a_r[1] + o_r[...]
        out_r[...] = jnp.dot(xx, w_r[...],
                             preferred_element_type=F32) + b_r[...]

    return pl.pallas_call(
        body, grid=(nr // BR,),
        in_specs=[pl.BlockSpec((NC, BR, D), lambda i: (0, i, 0)),
                  pl.BlockSpec((BR, D), lambda i: (i, 0)),
                  pl.BlockSpec((D, D), lambda i: (0, 0)),
                  pl.BlockSpec((1, D), lambda i: (0, 0))],
        out_specs=pl.BlockSpec((BR, D), lambda i: (i, 0)),
        out_shape=jax.ShapeDtypeStruct((nr, D), F32),
    )(aggp, o, p["W"], p["b"].reshape(1, D))


def _leaky(x):
    return jnp.where(x > 0, x, 0.01 * x)


def _mlp3_tc(x, ps):
    r = x.shape[0]
    nl = ps[2]["W"].shape[1]
    br = min(BR, r)

    def body(x_r, w1, b1, w2, b2, w3, b3, o_r):
        h = _leaky(jnp.dot(x_r[...], w1[...],
                           preferred_element_type=F32) + b1[...])
        h = _leaky(jnp.dot(h, w2[...], preferred_element_type=F32) + b2[...])
        o_r[...] = jnp.dot(h, w3[...], preferred_element_type=F32) + b3[...]

    wspec = pl.BlockSpec((D, D), lambda i: (0, 0))
    bspec = pl.BlockSpec((1, D), lambda i: (0, 0))
    return pl.pallas_call(
        body, grid=(r // br,),
        in_specs=[pl.BlockSpec((br, D), lambda i: (i, 0)),
                  wspec, bspec, wspec, bspec,
                  pl.BlockSpec((D, nl), lambda i: (0, 0)),
                  pl.BlockSpec((1, nl), lambda i: (0, 0))],
        out_specs=pl.BlockSpec((br, nl), lambda i: (i, 0)),
        out_shape=jax.ShapeDtypeStruct((r, nl), F32),
    )(x, ps[0]["W"], ps[0]["b"].reshape(1, D),
      ps[1]["W"], ps[1]["b"].reshape(1, D),
      ps[2]["W"], ps[2]["b"].reshape(1, nl))


def _poolfin_tc(sums, cnts):
    def body(sm_r, cn_r, o_r):
        sm = sm_r[0, :B, :] + sm_r[1, :B, :]
        cnt = jnp.sum(cn_r[...], axis=0)[:B]
        o_r[...] = sm / jnp.maximum(cnt, 1.0)[:, None]

    return pl.pallas_call(
        body,
        out_shape=jax.ShapeDtypeStruct((B, D), F32),
    )(sums, cnts)


def _pad_rows(a, rows):
    return jnp.pad(a, ((0, rows - a.shape[0]), (0, 0)))


def _pad_idx(a, n, fill):
    return jnp.concatenate(
        [a.astype(I32), jnp.full((n - a.shape[0],), fill, I32)])


def kernel(x, edge_attr, cond, params, edge_index, batch, non_edge_index):
    xp = _pad_rows(x, NR)
    eap = _pad_rows(edge_attr, EP)
    src = _pad_idx(edge_index[0], EP, DN)
    dst = _pad_idx(edge_index[1], EP, DN)

    z2_nr = jnp.zeros((NR, D), F32)
    z1_nr = jnp.zeros((NR,), F32)
    ninf_nr = jnp.full((NR,), -jnp.inf, F32)
    z2_anr = jnp.zeros((ANR, D), F32)
    z1_anr = jnp.zeros((ANR,), F32)
    ninf_anr = jnp.full((ANR,), -jnp.inf, F32)
    z2_sr = jnp.zeros((SR, D), F32)
    z1_sr = jnp.zeros((SR,), F32)

    o = _lin_tc(xp, params["x2h"]["W"], params["x2h"]["b"])
    e = _lin_tc(eap, params["e2h"]["W"], params["e2h"]["b"])
    c = _lin_tc(cond, params["c2h"]["W"], params["c2h"]["b"])

    gather3 = _build_gather3(EP, NR)
    segmax = _build_segmax(EP, NR)
    tscat = _build_tscatter(EP, NR, True)
    gscat = _build_gscatter(EP, NR)

    for i in range(6):
        tp = params["tconv"][i]
        q, k, v, sk = _qkvs_tc(o, tp)
        ee = _lin_tc(e, tp["edge"]["W"], tp["edge"]["b"])
        qd, ks, vs = gather3(q, k, v, src, dst)
        lg = _logits_tc(qd, ks, ee)
        mp = segmax(lg, dst, ninf_nr)
        m = _maxred_tc(mp)
        aggp, sp = tscat(lg, dst, m, vs, ee, z2_nr, z1_nr)
        ot = _combine_tc(aggp, sp, sk)
        gaggp = gscat(ot, e, src, dst, z2_nr)
        o = _gencomb_tc(gaggp, ot, params["gen"][i])

    u = jnp.arange(N, dtype=I32)
    vv = batch.astype(I32) + N
    asrc = _pad_idx(jnp.concatenate([edge_index[0].astype(I32), u, vv]),
                    EAP, ADN)
    adst = _pad_idx(jnp.concatenate([edge_index[1].astype(I32), vv, u]),
                    EAP, ADN)
    o2 = _pad_rows(jnp.concatenate([o[:N], c], axis=0), ANR)

    gather3a = _build_gather3(EAP, ANR)
    segmaxa = _build_segmax(EAP, ANR)
    tscata = _build_tscatter(EAP, ANR, False)

    for j in range(2):
        tp = params["o2o"][j]
        q, k, v, sk = _qkvs_tc(o2, tp)
        qd, ks, vs = gather3a(q, k, v, asrc, adst)
        lg = _logits_tc(qd, ks, None)
        mp = segmaxa(lg, adst, ninf_anr)
        m = _maxred_tc(mp)
        aggp, sp = tscata(lg, adst, m, vs, z2_anr, z1_anr)
        o2 = _combine_tc(aggp, sp, sk, o2)

    pidx = jnp.concatenate([batch.astype(I32),
                            jnp.arange(B, dtype=I32),
                            jnp.full((ANR - N - B,), B, I32)])
    sums, cnts = _build_pool(ANR)(o2, pidx, z2_sr, z1_sr)
    glob = _poolfin_tc(sums, cnts)

    o_f = _pad_rows(o2[:N], NR)
    stop = _mlp3_tc(glob, params["emb2stop"])
    add_node = _mlp3_tc(o_f, params["emb2add_node"])[:N]
    set_node_attr = _mlp3_tc(o_f, params["emb2set_node_attr"])[:N]

    ner = _pad_idx(non_edge_index[0], NEP, DN)
    nec = _pad_idx(non_edge_index[1], NEP, DN)
    pe = _build_pairadd(NEP, NR)(o_f, ner, nec)
    add_edge = _mlp3_tc(pe, params["emb2add_edge"])[:NE]

    ser = _pad_idx(edge_index[0, ::2], ESP, DN)
    sec = _pad_idx(edge_index[1, ::2], ESP, DN)
    se = _build_pairadd(ESP, NR)(o_f, ser, sec)
    set_edge_attr = _mlp3_tc(se, params["emb2set_edge_attr"])[:E // 2]

    reward = _mlp3_tc(glob, params["emb2reward"])

    return (stop, add_node, set_node_attr, add_edge, set_edge_attr, reward)


# speedup vs baseline: 2.7486x; 2.7486x over previous
import functools

import jax
import jax.numpy as jnp
from jax import lax
from jax.experimental import pallas as pl
from jax.experimental.pallas import tpu as pltpu
from jax.experimental.pallas import tpu_sc as plsc

F32 = jnp.float32
I32 = jnp.int32
D = 64
NC, NS, NW = 2, 16, 32
EB = 128
BR = 512

N = 10000
E = 160000
NE = 100000
B = 256

NR = 10240
DN = 10000
ANR = 10752
ADN = 10256
EP = 163840
EAP = 184320
NEP = 102400
ESP = 81920
SR = 272


def _mesh():
    return plsc.VectorSubcoreMesh(core_axis_name="c", subcore_axis_name="s")


_SC_PARAMS = pltpu.CompilerParams(use_tc_tiling_on_sc=False, needs_layout_passes=False)


def _wid():
    return lax.axis_index("s") * NC + lax.axis_index("c")


@functools.lru_cache(None)
def _build_gather3(ep, nr):
    pt = ep // NW
    nb = pt // EB

    def body(q_h, k_h, v_h, src_h, dst_h, qd_h, ks_h, vs_h,
             is_v, id_v, bq, bk, bv, s1, s2, s3):
        base = _wid() * pt

        def blk(b, carry):
            off = base + b * EB
            pltpu.sync_copy(src_h.at[pl.ds(off, EB)], is_v)
            pltpu.sync_copy(dst_h.at[pl.ds(off, EB)], id_v)
            d1 = pltpu.async_copy(q_h.at[id_v], bq, s1)
            d2 = pltpu.async_copy(k_h.at[is_v], bk, s2)
            d3 = pltpu.async_copy(v_h.at[is_v], bv, s3)
            d1.wait()
            d2.wait()
            d3.wait()
            w1 = pltpu.async_copy(bq, qd_h.at[pl.ds(off, EB)], s1)
            w2 = pltpu.async_copy(bk, ks_h.at[pl.ds(off, EB)], s2)
            w3 = pltpu.async_copy(bv, vs_h.at[pl.ds(off, EB)], s3)
            w1.wait()
            w2.wait()
            w3.wait()
            return carry

        lax.fori_loop(0, nb, blk, 0)

    return pl.kernel(
        body,
        out_type=(jax.ShapeDtypeStruct((ep, D), F32),) * 3,
        mesh=_mesh(),
        compiler_params=_SC_PARAMS,
        scratch_types=[
            pltpu.VMEM((EB,), I32), pltpu.VMEM((EB,), I32),
            pltpu.VMEM((EB, D), F32), pltpu.VMEM((EB, D), F32),
            pltpu.VMEM((EB, D), F32),
            pltpu.SemaphoreType.DMA, pltpu.SemaphoreType.DMA,
            pltpu.SemaphoreType.DMA,
        ],
        name=f"sc_gather3_{ep}",
    )


@functools.lru_cache(None)
def _build_segmax(ep, nr):
    pt = ep // NW
    nb = pt // EB

    def body(lg_h, dst_h, ninf_h, mp_h, m_loc, id_v, lg_v):
        w = _wid()
        pltpu.sync_copy(ninf_h, m_loc)
        base = w * pt

        iota = lax.iota(I32, 16)

        def blk(b, carry):
            off = base + b * EB
            pltpu.sync_copy(dst_h.at[pl.ds(off, EB)], id_v)
            pltpu.sync_copy(lg_h.at[pl.ds(off, EB)], lg_v)
            for g in range(EB // 16):
                d16 = id_v[pl.ds(g * 16, 16)]
                l16 = lg_v[pl.ds(g * 16, 16)]
                sd, sv = plsc.sort_key_val(d16, l16)
                for sh in (1, 2, 4, 8):
                    pidx = jnp.maximum(iota - sh, 0)
                    pk = sd.at[pidx].get(mode="promise_in_bounds")
                    pv = sv.at[pidx].get(mode="promise_in_bounds")
                    ok = (pk == sd) & (iota >= sh)
                    sv = jnp.where(ok, jnp.maximum(sv, pv), sv)
                nidx = jnp.minimum(iota + 1, 15)
                nk = sd.at[nidx].get(mode="promise_in_bounds")
                is_last = (nk != sd) | (iota == 15)
                cur = plsc.load_gather(m_loc, [sd])
                plsc.store_scatter(m_loc, [sd], jnp.maximum(cur, sv),
                                   mask=is_last)
            return carry

        lax.fori_loop(0, nb, blk, 0)
        pltpu.sync_copy(m_loc, mp_h.at[w])

    return pl.kernel(
        body,
        out_type=jax.ShapeDtypeStruct((NW, nr), F32),
        mesh=_mesh(),
        compiler_params=_SC_PARAMS,
        scratch_types=[
            pltpu.VMEM((nr,), F32), pltpu.VMEM((EB,), I32),
            pltpu.VMEM((EB,), F32),
        ],
        name=f"sc_segmax_{ep}",
    )


@functools.lru_cache(None)
def _build_tscatter(ep, nr, has_ee):
    pt = ep // NW
    nb = pt // EB
    zsl = nr // NS

    def body(*refs):
        if has_ee:
            (lg_h, dst_h, m_h, vs_h, ee_h, z1_h, agg_h, sp_h,
             agg_sh, m_loc, s_loc, id_v, lg_v, bv, be, bo, zb, s1, s2) = refs
        else:
            (lg_h, dst_h, m_h, vs_h, z1_h, agg_h, sp_h,
             agg_sh, m_loc, s_loc, id_v, lg_v, bv, bo, zb, s1, s2) = refs
            be = None
        c = lax.axis_index("c")
        s = lax.axis_index("s")
        w = s * NC + c
        pltpu.sync_copy(m_h, m_loc)
        pltpu.sync_copy(z1_h, s_loc)
        zero16 = jnp.zeros((16,), F32)
        for r in range(64):
            for chh in range(4):
                zb[r, pl.ds(chh * 16, 16)] = zero16
        for t in range(zsl // 64):
            pltpu.sync_copy(zb, agg_sh.at[pl.ds(s * zsl + t * 64, 64)])
        if zsl % 64:
            pltpu.sync_copy(zb.at[pl.ds(0, zsl % 64)],
                            agg_sh.at[pl.ds(s * zsl + (zsl // 64) * 64,
                                            zsl % 64)])
        plsc.subcore_barrier()
        base = w * pt

        def blk(b, carry):
            off = base + b * EB
            pltpu.sync_copy(dst_h.at[pl.ds(off, EB)], id_v)
            pltpu.sync_copy(lg_h.at[pl.ds(off, EB)], lg_v)
            d1 = pltpu.async_copy(vs_h.at[pl.ds(off, EB)], bv, s1)
            if has_ee:
                d2 = pltpu.async_copy(ee_h.at[pl.ds(off, EB)], be, s2)
            d1.wait()
            if has_ee:
                d2.wait()
            for g in range(EB // 16):
                d16 = id_v[pl.ds(g * 16, 16)]
                l16 = lg_v[pl.ds(g * 16, 16)]
                mv = plsc.load_gather(m_loc, [d16])
                ex = jnp.exp(l16 - mv)
                plsc.addupdate_scatter(s_loc, [d16], ex)
                for j in range(16):
                    ei = g * 16 + j
                    exj = ex[j]
                    for ch in range(4):
                        sl = pl.ds(ch * 16, 16)
                        vv = bv[ei, sl]
                        if has_ee:
                            vv = vv + be[ei, sl]
                        bo[ei, sl] = vv * exj
            pltpu.sync_copy(bo, agg_sh.at[id_v], add=True)
            return carry

        lax.fori_loop(0, nb, blk, 0)
        plsc.subcore_barrier()
        nch = zsl // 64 + (1 if zsl % 64 else 0)
        for t in range(nch):
            rows = 64 if (t + 1) * 64 <= zsl else zsl % 64
            pltpu.sync_copy(agg_sh.at[pl.ds(s * zsl + t * 64, rows)],
                            zb.at[pl.ds(0, rows)])
            pltpu.sync_copy(zb.at[pl.ds(0, rows)],
                            agg_h.at[c, pl.ds(s * zsl + t * 64, rows)])
        pltpu.sync_copy(s_loc, sp_h.at[w])

    scratch = [
        pltpu.VMEM_SHARED((nr, D), F32),
        pltpu.VMEM((nr,), F32), pltpu.VMEM((nr,), F32),
        pltpu.VMEM((EB,), I32), pltpu.VMEM((EB,), F32),
        pltpu.VMEM((EB, D), F32),
    ]
    if has_ee:
        scratch.append(pltpu.VMEM((EB, D), F32))
    scratch.append(pltpu.VMEM((EB, D), F32))
    scratch.append(pltpu.VMEM((64, D), F32))
    scratch += [pltpu.SemaphoreType.DMA, pltpu.SemaphoreType.DMA]
    return pl.kernel(
        body,
        out_type=(jax.ShapeDtypeStruct((NC, nr, D), F32),
                  jax.ShapeDtypeStruct((NW, nr), F32)),
        mesh=_mesh(),
        compiler_params=_SC_PARAMS,
        scratch_types=scratch,
        name=f"sc_tscatter_{ep}_{int(has_ee)}",
    )


@functools.lru_cache(None)
def _build_gscatter(ep, nr):
    pt = ep // NW
    nb = pt // EB
    zsl = nr // NS

    def body(o_h, e_h, src_h, dst_h, agg_h,
             agg_sh, is_v, id_v, bo, be, bm, zb, s1, s2):
        c = lax.axis_index("c")
        s = lax.axis_index("s")
        w = s * NC + c
        zero16 = jnp.zeros((16,), F32)
        for r in range(64):
            for chh in range(4):
                zb[r, pl.ds(chh * 16, 16)] = zero16
        for t in range(zsl // 64):
            pltpu.sync_copy(zb, agg_sh.at[pl.ds(s * zsl + t * 64, 64)])
        if zsl % 64:
            pltpu.sync_copy(zb.at[pl.ds(0, zsl % 64)],
                            agg_sh.at[pl.ds(s * zsl + (zsl // 64) * 64,
                                            zsl % 64)])
        plsc.subcore_barrier()
        base = w * pt

        def blk(b, carry):
            off = base + b * EB
            pltpu.sync_copy(src_h.at[pl.ds(off, EB)], is_v)
            pltpu.sync_copy(dst_h.at[pl.ds(off, EB)], id_v)
            d1 = pltpu.async_copy(o_h.at[is_v], bo, s1)
            d2 = pltpu.async_copy(e_h.at[pl.ds(off, EB)], be, s2)
            d1.wait()
            d2.wait()
            for r in range(EB):
                for ch in range(4):
                    sl = pl.ds(ch * 16, 16)
                    bm[r, sl] = jnp.maximum(bo[r, sl] + be[r, sl], 0.0) + 1e-7
            pltpu.sync_copy(bm, agg_sh.at[id_v], add=True)
            return carry

        lax.fori_loop(0, nb, blk, 0)
        plsc.subcore_barrier()
        nch = zsl // 64 + (1 if zsl % 64 else 0)
        for t in range(nch):
            rows = 64 if (t + 1) * 64 <= zsl else zsl % 64
            pltpu.sync_copy(agg_sh.at[pl.ds(s * zsl + t * 64, rows)],
                            zb.at[pl.ds(0, rows)])
            pltpu.sync_copy(zb.at[pl.ds(0, rows)],
                            agg_h.at[c, pl.ds(s * zsl + t * 64, rows)])

    return pl.kernel(
        body,
        out_type=jax.ShapeDtypeStruct((NC, nr, D), F32),
        mesh=_mesh(),
        compiler_params=_SC_PARAMS,
        scratch_types=[
            pltpu.VMEM_SHARED((nr, D), F32),
            pltpu.VMEM((EB,), I32), pltpu.VMEM((EB,), I32),
            pltpu.VMEM((EB, D), F32), pltpu.VMEM((EB, D), F32),
            pltpu.VMEM((EB, D), F32), pltpu.VMEM((64, D), F32),
            pltpu.SemaphoreType.DMA, pltpu.SemaphoreType.DMA,
        ],
        name=f"sc_gscatter_{ep}",
    )


@functools.lru_cache(None)
def _build_pairadd(ep, nr):
    pt = ep // NW
    nb = pt // EB

    def body(t_h, ia_h, ib_h, out_h, ia_v, ib_v, ba, bb, s1, s2):
        base = _wid() * pt

        def blk(b, carry):
            off = base + b * EB
            pltpu.sync_copy(ia_h.at[pl.ds(off, EB)], ia_v)
            pltpu.sync_copy(ib_h.at[pl.ds(off, EB)], ib_v)
            d1 = pltpu.async_copy(t_h.at[ia_v], ba, s1)
            d2 = pltpu.async_copy(t_h.at[ib_v], bb, s2)
            d1.wait()
            d2.wait()
            for r in range(EB):
                for ch in range(4):
                    sl = pl.ds(ch * 16, 16)
                    ba[r, sl] = ba[r, sl] + bb[r, sl]
            pltpu.sync_copy(ba, out_h.at[pl.ds(off, EB)])
            return carry

        lax.fori_loop(0, nb, blk, 0)

    return pl.kernel(
        body,
        out_type=jax.ShapeDtypeStruct((ep, D), F32),
        mesh=_mesh(),
        compiler_params=_SC_PARAMS,
        scratch_types=[
            pltpu.VMEM((EB,), I32), pltpu.VMEM((EB,), I32),
            pltpu.VMEM((EB, D), F32), pltpu.VMEM((EB, D), F32),
            pltpu.SemaphoreType.DMA, pltpu.SemaphoreType.DMA,
        ],
        name=f"sc_pairadd_{ep}",
    )


@functools.lru_cache(None)
def _build_pool(pr):
    pt = pr // NW
    pb = 112
    nb = pt // pb
    zsl = SR // NS

    def body(rows_h, pidx_h, z1_h, sums_h, cnt_h,
             sums_sh, cnt_loc, ip_v, br_v, zb, s1):
        c = lax.axis_index("c")
        s = lax.axis_index("s")
        w = s * NC + c
        pltpu.sync_copy(z1_h, cnt_loc)
        zero16 = jnp.zeros((16,), F32)
        for r in range(zsl):
            for chh in range(4):
                zb[r, pl.ds(chh * 16, 16)] = zero16
        pltpu.sync_copy(zb, sums_sh.at[pl.ds(s * zsl, zsl)])
        plsc.subcore_barrier()
        base = w * pt
        ones = jnp.ones((16,), F32)

        def blk(b, carry):
            off = base + b * pb
            pltpu.sync_copy(pidx_h.at[pl.ds(off, pb)], ip_v)
            d1 = pltpu.async_copy(rows_h.at[pl.ds(off, pb)], br_v, s1)
            d1.wait()
            for g in range(pb // 16):
                p16 = ip_v[pl.ds(g * 16, 16)]
                plsc.addupdate_scatter(cnt_loc, [p16], ones)
            pltpu.sync_copy(br_v, sums_sh.at[ip_v], add=True)
            return carry

        lax.fori_loop(0, nb, blk, 0)
        plsc.subcore_barrier()
        pltpu.sync_copy(sums_sh.at[pl.ds(s * zsl, zsl)], zb)
        pltpu.sync_copy(zb, sums_h.at[c, pl.ds(s * zsl, zsl)])
        pltpu.sync_copy(cnt_loc, cnt_h.at[w])

    return pl.kernel(
        body,
        out_type=(jax.ShapeDtypeStruct((NC, SR, D), F32),
                  jax.ShapeDtypeStruct((NW, SR), F32)),
        mesh=_mesh(),
        compiler_params=_SC_PARAMS,
        scratch_types=[
            pltpu.VMEM_SHARED((SR, D), F32),
            pltpu.VMEM((SR,), F32), pltpu.VMEM((112,), I32),
            pltpu.VMEM((112, D), F32), pltpu.VMEM((17, D), F32),
            pltpu.SemaphoreType.DMA,
        ],
        name="sc_pool",
    )


def _lin_tc(x, w, b):
    r, din = x.shape
    dout = w.shape[1]
    br = min(BR, r)

    def body(x_r, w_r, b_r, o_r):
        o_r[...] = jnp.dot(x_r[...], w_r[...],
                           preferred_element_type=F32) + b_r[...]

    return pl.pallas_call(
        body, grid=(r // br,),
        in_specs=[pl.BlockSpec((br, din), lambda i: (i, 0)),
                  pl.BlockSpec((din, dout), lambda i: (0, 0)),
                  pl.BlockSpec((1, dout), lambda i: (0, 0))],
        out_specs=pl.BlockSpec((br, dout), lambda i: (i, 0)),
        out_shape=jax.ShapeDtypeStruct((r, dout), F32),
    )(x, w, b.reshape(1, -1))


def _qkvs_tc(o, p):
    r = o.shape[0]

    def body(o_r, wq, bq, wk, bk, wv, bv, ws, bs, q_o, k_o, v_o, s_o):
        ob = o_r[...]
        q_o[...] = jnp.dot(ob, wq[...], preferred_element_type=F32) + bq[...]
        k_o[...] = jnp.dot(ob, wk[...], preferred_element_type=F32) + bk[...]
        v_o[...] = jnp.dot(ob, wv[...], preferred_element_type=F32) + bv[...]
        s_o[...] = jnp.dot(ob, ws[...], preferred_element_type=F32) + bs[...]

    wspec = pl.BlockSpec((D, D), lambda i: (0, 0))
    bspec = pl.BlockSpec((1, D), lambda i: (0, 0))
    rspec = pl.BlockSpec((BR, D), lambda i: (i, 0))
    return pl.pallas_call(
        body, grid=(r // BR,),
        in_specs=[rspec, wspec, bspec, wspec, bspec, wspec, bspec, wspec,
                  bspec],
        out_specs=[rspec] * 4,
        out_shape=[jax.ShapeDtypeStruct((r, D), F32)] * 4,
    )(o, p["q"]["W"], p["q"]["b"].reshape(1, D),
      p["k"]["W"], p["k"]["b"].reshape(1, D),
      p["v"]["W"], p["v"]["b"].reshape(1, D),
      p["skip"]["W"], p["skip"]["b"].reshape(1, D))


def _logits_tc(qd, ks, ee):
    r = qd.shape[0]
    if ee is None:
        def body(q_r, k_r, o_r):
            o_r[...] = jnp.sum(q_r[...] * k_r[...], axis=1,
                               keepdims=True) * 0.125
        args = (qd, ks)
        nin = 2
    else:
        def body(q_r, k_r, e_r, o_r):
            o_r[...] = jnp.sum(q_r[...] * (k_r[...] + e_r[...]), axis=1,
                               keepdims=True) * 0.125
        args = (qd, ks, ee)
        nin = 3
    out = pl.pallas_call(
        body, grid=(r // BR,),
        in_specs=[pl.BlockSpec((BR, D), lambda i: (i, 0))] * nin,
        out_specs=pl.BlockSpec((BR, 1), lambda i: (i, 0)),
        out_shape=jax.ShapeDtypeStruct((r, 1), F32),
    )(*args)
    return out.reshape(r)


def _maxred_tc(mp):
    nw, nr = mp.shape

    def body(m_r, o_r):
        o_r[...] = jnp.max(m_r[...], axis=0, keepdims=True)

    out = pl.pallas_call(
        body, grid=(nr // BR,),
        in_specs=[pl.BlockSpec((nw, BR), lambda i: (0, i))],
        out_specs=pl.BlockSpec((1, BR), lambda i: (0, i)),
        out_shape=jax.ShapeDtypeStruct((1, nr), F32),
    )(mp)
    return out.reshape(nr)


def _combine_tc(aggp, sp, sk, o_prev=None):
    nr = sk.shape[0]
    resid = o_prev is not None

    def body(*refs):
        if resid:
            a_r, s_r, k_r, p_r, o_r = refs
        else:
            a_r, s_r, k_r, o_r = refs
        agg = a_r[0] + a_r[1]
        s = jnp.sum(s_r[...], axis=0)
        res = agg / (s[:, None] + 1e-16) + k_r[...]
        if resid:
            res = res + p_r[...]
        o_r[...] = res

    specs = [pl.BlockSpec((NC, BR, D), lambda i: (0, i, 0)),
             pl.BlockSpec((NW, BR), lambda i: (0, i)),
             pl.BlockSpec((BR, D), lambda i: (i, 0))]
    args = [aggp, sp, sk]
    if resid:
        specs.append(pl.BlockSpec((BR, D), lambda i: (i, 0)))
        args.append(o_prev)
    return pl.pallas_call(
        body, grid=(nr // BR,),
        in_specs=specs,
        out_specs=pl.BlockSpec((BR, D), lambda i: (i, 0)),
        out_shape=jax.ShapeDtypeStruct((nr, D), F32),
    )(*args)


def _gencomb_tc(aggp, o, p):
    nr = o.shape[0]

    def body(a_r, o_r, w_r, b_r, out_r):
        xx = a_r[0] + a_r[1] + o_r[...]
        out_r[...] = jnp.dot(xx, w_r[...],
                             preferred_element_type=F32) + b_r[...]

    return pl.pallas_call(
        body, grid=(nr // BR,),
        in_specs=[pl.BlockSpec((NC, BR, D), lambda i: (0, i, 0)),
                  pl.BlockSpec((BR, D), lambda i: (i, 0)),
                  pl.BlockSpec((D, D), lambda i: (0, 0)),
                  pl.BlockSpec((1, D), lambda i: (0, 0))],
        out_specs=pl.BlockSpec((BR, D), lambda i: (i, 0)),
        out_shape=jax.ShapeDtypeStruct((nr, D), F32),
    )(aggp, o, p["W"], p["b"].reshape(1, D))


def _leaky(x):
    return jnp.where(x > 0, x, 0.01 * x)


def _mlp3_tc(x, ps):
    r = x.shape[0]
    nl = ps[2]["W"].shape[1]
    br = min(BR, r)

    def body(x_r, w1, b1, w2, b2, w3, b3, o_r):
        h = _leaky(jnp.dot(x_r[...], w1[...],
                           preferred_element_type=F32) + b1[...])
        h = _leaky(jnp.dot(h, w2[...], preferred_element_type=F32) + b2[...])
        o_r[...] = jnp.dot(h, w3[...], preferred_element_type=F32) + b3[...]

    wspec = pl.BlockSpec((D, D), lambda i: (0, 0))
    bspec = pl.BlockSpec((1, D), lambda i: (0, 0))
    return pl.pallas_call(
        body, grid=(r // br,),
        in_specs=[pl.BlockSpec((br, D), lambda i: (i, 0)),
                  wspec, bspec, wspec, bspec,
                  pl.BlockSpec((D, nl), lambda i: (0, 0)),
                  pl.BlockSpec((1, nl), lambda i: (0, 0))],
        out_specs=pl.BlockSpec((br, nl), lambda i: (i, 0)),
        out_shape=jax.ShapeDtypeStruct((r, nl), F32),
    )(x, ps[0]["W"], ps[0]["b"].reshape(1, D),
      ps[1]["W"], ps[1]["b"].reshape(1, D),
      ps[2]["W"], ps[2]["b"].reshape(1, nl))


def _poolfin_tc(sums, cnts):
    def body(sm_r, cn_r, o_r):
        sm = sm_r[0, :B, :] + sm_r[1, :B, :]
        cnt = jnp.sum(cn_r[...], axis=0)[:B]
        o_r[...] = sm / jnp.maximum(cnt, 1.0)[:, None]

    return pl.pallas_call(
        body,
        out_shape=jax.ShapeDtypeStruct((B, D), F32),
    )(sums, cnts)


def _pad_rows(a, rows):
    return jnp.pad(a, ((0, rows - a.shape[0]), (0, 0)))


def _pad_idx(a, n, fill):
    return jnp.concatenate(
        [a.astype(I32), jnp.full((n - a.shape[0],), fill, I32)])


def kernel(x, edge_attr, cond, params, edge_index, batch, non_edge_index):
    xp = _pad_rows(x, NR)
    eap = _pad_rows(edge_attr, EP)
    src = _pad_idx(edge_index[0], EP, DN)
    dst = _pad_idx(edge_index[1], EP, DN)

    z1_nr = jnp.zeros((NR,), F32)
    ninf_nr = jnp.full((NR,), -jnp.inf, F32)
    z1_anr = jnp.zeros((ANR,), F32)
    ninf_anr = jnp.full((ANR,), -jnp.inf, F32)
    z1_sr = jnp.zeros((SR,), F32)

    o = _lin_tc(xp, params["x2h"]["W"], params["x2h"]["b"])
    e = _lin_tc(eap, params["e2h"]["W"], params["e2h"]["b"])
    c = _lin_tc(cond, params["c2h"]["W"], params["c2h"]["b"])

    gather3 = _build_gather3(EP, NR)
    segmax = _build_segmax(EP, NR)
    tscat = _build_tscatter(EP, NR, True)
    gscat = _build_gscatter(EP, NR)

    for i in range(6):
        tp = params["tconv"][i]
        q, k, v, sk = _qkvs_tc(o, tp)
        ee = _lin_tc(e, tp["edge"]["W"], tp["edge"]["b"])
        qd, ks, vs = gather3(q, k, v, src, dst)
        lg = _logits_tc(qd, ks, ee)
        mp = segmax(lg, dst, ninf_nr)
        m = _maxred_tc(mp)
        aggp, sp = tscat(lg, dst, m, vs, ee, z1_nr)
        ot = _combine_tc(aggp, sp, sk)
        gaggp = gscat(ot, e, src, dst)
        o = _gencomb_tc(gaggp, ot, params["gen"][i])

    u = jnp.arange(N, dtype=I32)
    vv = batch.astype(I32) + N
    asrc = _pad_idx(jnp.concatenate([edge_index[0].astype(I32), u, vv]),
                    EAP, ADN)
    adst = _pad_idx(jnp.concatenate([edge_index[1].astype(I32), vv, u]),
                    EAP, ADN)
    o2 = _pad_rows(jnp.concatenate([o[:N], c], axis=0), ANR)

    gather3a = _build_gather3(EAP, ANR)
    segmaxa = _build_segmax(EAP, ANR)
    tscata = _build_tscatter(EAP, ANR, False)

    for j in range(2):
        tp = params["o2o"][j]
        q, k, v, sk = _qkvs_tc(o2, tp)
        qd, ks, vs = gather3a(q, k, v, asrc, adst)
        lg = _logits_tc(qd, ks, None)
        mp = segmaxa(lg, adst, ninf_anr)
        m = _maxred_tc(mp)
        aggp, sp = tscata(lg, adst, m, vs, z1_anr)
        o2 = _combine_tc(aggp, sp, sk, o2)

    pidx = jnp.concatenate([batch.astype(I32),
                            jnp.arange(B, dtype=I32),
                            jnp.full((ANR - N - B,), B, I32)])
    sums, cnts = _build_pool(ANR)(o2, pidx, z1_sr)
    glob = _poolfin_tc(sums, cnts)

    o_f = _pad_rows(o2[:N], NR)
    stop = _mlp3_tc(glob, params["emb2stop"])
    add_node = _mlp3_tc(o_f, params["emb2add_node"])[:N]
    set_node_attr = _mlp3_tc(o_f, params["emb2set_node_attr"])[:N]

    ner = _pad_idx(non_edge_index[0], NEP, DN)
    nec = _pad_idx(non_edge_index[1], NEP, DN)
    pe = _build_pairadd(NEP, NR)(o_f, ner, nec)
    add_edge = _mlp3_tc(pe, params["emb2add_edge"])[:NE]

    ser = _pad_idx(edge_index[0, ::2], ESP, DN)
    sec = _pad_idx(edge_index[1, ::2], ESP, DN)
    se = _build_pairadd(ESP, NR)(o_f, ser, sec)
    set_edge_attr = _mlp3_tc(se, params["emb2set_edge_attr"])[:E // 2]

    reward = _mlp3_tc(glob, params["emb2reward"])

    return (stop, add_node, set_node_attr, add_edge, set_edge_attr, reward)
